# baseline (speedup 1.0000x reference)
import functools

import jax
import jax.numpy as jnp
from jax import lax
from jax.experimental import pallas as pl
from jax.experimental.pallas import tpu as pltpu
from jax.experimental.pallas import tpu_sc as plsc

NC = 2
NS = 16
L = 16


def _gather_body(v_hbm, vb_hbm, out_hbm, vb, buf, sem, *, K, BT):
    wid = lax.axis_index("s") * NC + lax.axis_index("c")
    b0 = wid * BT

    pltpu.sync_copy(vb_hbm, vb)

    def _copy(k):
        vk = jnp.max(vb[k, pl.ds(0, L)])
        return pltpu.make_async_copy(
            v_hbm.at[pl.ds(b0, BT), vk], buf.at[pl.ds(0, BT), k], sem
        )

    def start_body(k, carry):
        _copy(k).start()
        return carry

    def wait_body(k, carry):
        _copy(k).wait()
        return carry

    half = K // 2
    for lo, hi in ((0, half), (half, K)):
        lax.fori_loop(lo, hi, start_body, 0)
        lax.fori_loop(lo, hi, wait_body, 0)

    pltpu.sync_copy(buf, out_hbm.at[pl.ds(b0, BT)])


def kernel(vertices, vert_idx):
    B, V, C = vertices.shape
    (K,) = vert_idx.shape
    NW = NC * NS
    BT = B // NW
    assert B % NW == 0

    vidxb = jnp.repeat(vert_idx.astype(jnp.int32)[:, None], L, axis=1)

    mesh = plsc.VectorSubcoreMesh(
        core_axis_name="c", subcore_axis_name="s",
        num_cores=NC, num_subcores=NS,
    )
    return pl.kernel(
        functools.partial(_gather_body, K=K, BT=BT),
        out_type=jax.ShapeDtypeStruct((B, K, C), jnp.float32),
        mesh=mesh,
        scratch_types=[
            pltpu.VMEM((K, L), jnp.int32),
            pltpu.VMEM((BT, K, C), jnp.float32),
            pltpu.SemaphoreType.DMA,
        ],
        compiler_params=pltpu.CompilerParams(
            use_tc_tiling_on_sc=False, needs_layout_passes=False,
        ),
    )(vertices, vidxb)


# speedup vs baseline: 8.6401x; 8.6401x over previous
import functools

import jax
import jax.numpy as jnp
from jax.experimental import pallas as pl
from jax.experimental.pallas import tpu as pltpu


def _tc_body(vidx_s, v_any, o_any, sem, *, K):
    for k in range(K):
        pltpu.make_async_copy(v_any.at[:, vidx_s[k]], o_any.at[:, k], sem).start()
    for k in range(K):
        pltpu.make_async_copy(v_any.at[:, vidx_s[k]], o_any.at[:, k], sem).wait()


def kernel(vertices, vert_idx):
    B, V, C = vertices.shape
    (K,) = vert_idx.shape
    vidx = vert_idx.astype(jnp.int32)
    return pl.pallas_call(
        functools.partial(_tc_body, K=K),
        out_shape=jax.ShapeDtypeStruct((B, K, C), jnp.float32),
        in_specs=[pl.BlockSpec(memory_space=pltpu.SMEM),
                  pl.BlockSpec(memory_space=pl.ANY)],
        out_specs=pl.BlockSpec(memory_space=pl.ANY),
        scratch_shapes=[pltpu.SemaphoreType.DMA],
    )(vidx, vertices)
